# baseline (speedup 1.0000x reference)
import jax
import jax.numpy as jnp
from jax import lax
from jax.experimental import pallas as pl
from jax.experimental.pallas import tpu as pltpu

N = 100000
HIDDEN = 512
BLK = 4000
NBUF = 3
NBLK = N // BLK


def _tc_body(h_hbm, w_ref, b_ref, out_ref, buf, sems):
    i = pl.program_id(0)

    def start(block, slot):
        pltpu.make_async_copy(
            h_hbm.at[pl.ds(block * BLK, BLK), :],
            buf.at[slot],
            sems.at[slot],
        ).start()

    @pl.when(i == 0)
    def _prime():
        for k in range(NBUF):
            start(k, k)

    slot = lax.rem(i, NBUF)
    pltpu.make_async_copy(
        h_hbm.at[pl.ds(i * BLK, BLK), :], buf.at[slot], sems.at[slot]
    ).wait()

    h = jnp.maximum(buf[slot], 0.0)
    y = lax.dot_general(
        w_ref[...], h, (((0,), (1,)), ((), ())),
        preferred_element_type=jnp.float32,
    )
    out_ref[...] = (y + b_ref[0, 0]).reshape(1, 8, BLK // 8)

    nxt = i + NBUF

    @pl.when(nxt < NBLK)
    def _refill():
        start(nxt, slot)


def kernel(h_0, W_node, b_node):
    b2 = b_node.reshape(1, 1)
    out = pl.pallas_call(
        _tc_body,
        grid=(NBLK,),
        in_specs=[
            pl.BlockSpec(memory_space=pl.ANY),
            pl.BlockSpec((HIDDEN, 1), lambda i: (0, 0)),
            pl.BlockSpec(memory_space=pltpu.SMEM),
        ],
        out_specs=pl.BlockSpec((1, 8, BLK // 8), lambda i: (i, 0, 0)),
        out_shape=jax.ShapeDtypeStruct((NBLK, 8, BLK // 8), jnp.float32),
        scratch_shapes=[
            pltpu.VMEM((NBUF, BLK, HIDDEN), jnp.float32),
            pltpu.SemaphoreType.DMA((NBUF,)),
        ],
        compiler_params=pltpu.CompilerParams(
            dimension_semantics=("arbitrary",),
        ),
    )(h_0, W_node, b2)
    return out.reshape(N, 1)
